# Initial kernel scaffold; baseline (speedup 1.0000x reference)
import functools

import jax
import jax.numpy as jnp
from jax import lax
from jax.experimental import pallas as pl
from jax.experimental.pallas import tpu as pltpu
from jax.experimental.pallas import tpu_sc as plsc

N = 10000
E = 320000
D_IN = 128
D_H = 64

NC = 2
NS = 16
NW = NC * NS
EPW = E // NW
BATCH = 125
NCHUNK = EPW // BATCH
ROWS_PER_TILE = N // NS
ZROWS = 25

_mesh = plsc.VectorSubcoreMesh(core_axis_name="c", subcore_axis_name="s")


def _zero_fill(buf, nrows, ncols):
    zero = jnp.zeros((16,), jnp.float32)
    for r in range(nrows):
        for c in range(ncols // 16):
            buf[r, pl.ds(c * 16, 16)] = zero


def _spmm_sc(hp, srcr, dstr, F):
    @functools.partial(
        pl.kernel,
        mesh=_mesh,
        out_type=jax.ShapeDtypeStruct((2 * N, F), jnp.float32),
        scratch_types=[
            pltpu.VMEM((NCHUNK, BATCH), jnp.int32),
            pltpu.VMEM((NCHUNK, BATCH), jnp.int32),
            pltpu.VMEM((BATCH, F), jnp.float32),
            pltpu.VMEM((ZROWS, F), jnp.float32),
            pltpu.VMEM_SHARED((N, F), jnp.float32),
            pltpu.SemaphoreType.DMA,
        ],
    )
    def k(hp_hbm, src_hbm, dst_hbm, out_hbm, srcb, dstb, rows, zbuf, acc, sem):
        c = lax.axis_index("c")
        s = lax.axis_index("s")
        wid = c * NS + s
        _zero_fill(zbuf, ZROWS, F)
        for i in range(ROWS_PER_TILE // ZROWS):
            pltpu.sync_copy(zbuf, acc.at[pl.ds(s * ROWS_PER_TILE + i * ZROWS, ZROWS)])
        pltpu.sync_copy(src_hbm.at[wid], srcb)
        pltpu.sync_copy(dst_hbm.at[wid], dstb)
        plsc.subcore_barrier()

        def body(kk, carry):
            pltpu.async_copy(hp_hbm.at[srcb.at[kk]], rows, sem).wait()
            pltpu.sync_copy(rows, acc.at[dstb.at[kk]], add=True)
            return carry

        lax.fori_loop(0, NCHUNK, body, 0)
        plsc.subcore_barrier()
        pltpu.sync_copy(
            acc.at[pl.ds(s * ROWS_PER_TILE, ROWS_PER_TILE)],
            out_hbm.at[pl.ds(c * N + s * ROWS_PER_TILE, ROWS_PER_TILE)],
        )

    return k(hp, srcr, dstr)


def _deg_sc(dstr):
    F = 16

    @functools.partial(
        pl.kernel,
        mesh=_mesh,
        out_type=jax.ShapeDtypeStruct((2 * N, F), jnp.float32),
        scratch_types=[
            pltpu.VMEM((NCHUNK, BATCH), jnp.int32),
            pltpu.VMEM((BATCH, F), jnp.float32),
            pltpu.VMEM((ZROWS, F), jnp.float32),
            pltpu.VMEM_SHARED((N, F), jnp.float32),
        ],
    )
    def k(dst_hbm, out_hbm, dstb, ones, zbuf, acc):
        c = lax.axis_index("c")
        s = lax.axis_index("s")
        wid = c * NS + s
        _zero_fill(zbuf, ZROWS, F)
        one = jnp.ones((16,), jnp.float32)
        for r in range(BATCH):
            ones[r, pl.ds(0, 16)] = one
        for i in range(ROWS_PER_TILE // ZROWS):
            pltpu.sync_copy(zbuf, acc.at[pl.ds(s * ROWS_PER_TILE + i * ZROWS, ZROWS)])
        pltpu.sync_copy(dst_hbm.at[wid], dstb)
        plsc.subcore_barrier()

        def body(kk, carry):
            pltpu.sync_copy(ones, acc.at[dstb.at[kk]], add=True)
            return carry

        lax.fori_loop(0, NCHUNK, body, 0)
        plsc.subcore_barrier()
        pltpu.sync_copy(
            acc.at[pl.ds(s * ROWS_PER_TILE, ROWS_PER_TILE)],
            out_hbm.at[pl.ds(c * N + s * ROWS_PER_TILE, ROWS_PER_TILE)],
        )

    return k(dstr)


R = 1000
GRID = N // R


def _dinv_blk(degp):
    deg = degp[0, :, 0] + degp[1, :, 0] + 1.0
    return lax.rsqrt(deg)[:, None]


def _t0_body(x_ref, w1_ref, degp_ref, hp1_ref):
    dinv = _dinv_blk(degp_ref[...])
    h = jnp.dot(x_ref[...], w1_ref[...], preferred_element_type=jnp.float32)
    hp1_ref[...] = dinv * h


def _t1_body(s1_ref, hp1_ref, degp_ref, b1_ref, w2_ref, z_ref, hp2_ref):
    dinv = _dinv_blk(degp_ref[...])
    s = s1_ref[0] + s1_ref[1] + hp1_ref[...]
    z = jnp.maximum(dinv * s + b1_ref[...], 0.0)
    z_ref[...] = z
    hp2_ref[...] = dinv * jnp.dot(z, w2_ref[...], preferred_element_type=jnp.float32)


def _t2_body(s2_ref, hp2_ref, degp_ref, b2_ref, ws_ref, wa1_ref, hp34_ref):
    dinv = _dinv_blk(degp_ref[...])
    s = s2_ref[0] + s2_ref[1] + hp2_ref[...]
    z2 = jnp.maximum(dinv * s + b2_ref[...], 0.0)
    hp3 = dinv * jnp.dot(z2, ws_ref[...], preferred_element_type=jnp.float32)
    hp4 = dinv * jnp.dot(z2, wa1_ref[...], preferred_element_type=jnp.float32)
    hp34_ref[...] = jnp.concatenate([hp3, hp4], axis=1)


def _t3_body(s34_ref, hp34_ref, degp_ref, bs_ref, ba1_ref, wa2_ref, struz_ref, hp5_ref):
    dinv = _dinv_blk(degp_ref[...])
    s = s34_ref[0] + s34_ref[1] + hp34_ref[...]
    struz_ref[...] = jnp.maximum(dinv * s[:, :D_H] + bs_ref[...], 0.0)
    a = jnp.maximum(dinv * s[:, D_H:] + ba1_ref[...], 0.0)
    hp5_ref[...] = dinv * jnp.dot(a, wa2_ref[...], preferred_element_type=jnp.float32)


def _t4_body(s5_ref, hp5_ref, degp_ref, ba2_ref, attr_ref):
    dinv = _dinv_blk(degp_ref[...])
    s = s5_ref[0] + s5_ref[1] + hp5_ref[...]
    attr_ref[...] = jnp.maximum(dinv * s + ba2_ref[...], 0.0)


def _t5_body(zl_ref, zr_ref, out_ref):
    out_ref[...] = lax.dot_general(
        zl_ref[...], zr_ref[...], (((1,), (1,)), ((), ())),
        preferred_element_type=jnp.float32)


def _row_specs(widths):
    return [pl.BlockSpec((R, w), lambda i: (i, 0)) for w in widths]


def _part_spec(w):
    return pl.BlockSpec((2, R, w), lambda i: (0, i, 0))


def _full_spec(shape):
    return pl.BlockSpec(shape, lambda i: tuple(0 for _ in shape))


def kernel(x, edge_index, W_enc1, b_enc1, W_enc2, b_enc2, W_stru, b_stru,
           W_att1, b_att1, W_att2, b_att2):
    src = edge_index[0].reshape(NW, NCHUNK, BATCH)
    dst = edge_index[1].reshape(NW, NCHUNK, BATCH)

    degp = _deg_sc(dst).reshape(2, N, 16)

    hp1 = pl.pallas_call(
        _t0_body,
        grid=(GRID,),
        in_specs=[pl.BlockSpec((R, D_IN), lambda i: (i, 0)),
                  _full_spec((D_IN, D_H)), _part_spec(16)],
        out_specs=pl.BlockSpec((R, D_H), lambda i: (i, 0)),
        out_shape=jax.ShapeDtypeStruct((N, D_H), jnp.float32),
    )(x, W_enc1, degp)

    s1 = _spmm_sc(hp1, src, dst, D_H).reshape(2, N, D_H)

    z, hp2 = pl.pallas_call(
        _t1_body,
        grid=(GRID,),
        in_specs=[_part_spec(D_H), pl.BlockSpec((R, D_H), lambda i: (i, 0)),
                  _part_spec(16), _full_spec((1, D_H)), _full_spec((D_H, D_H))],
        out_specs=_row_specs([D_H, D_H]),
        out_shape=[jax.ShapeDtypeStruct((N, D_H), jnp.float32),
                   jax.ShapeDtypeStruct((N, D_H), jnp.float32)],
    )(s1, hp1, degp, b_enc1.reshape(1, D_H), W_enc2)

    s2 = _spmm_sc(hp2, src, dst, D_H).reshape(2, N, D_H)

    hp34 = pl.pallas_call(
        _t2_body,
        grid=(GRID,),
        in_specs=[_part_spec(D_H), pl.BlockSpec((R, D_H), lambda i: (i, 0)),
                  _part_spec(16), _full_spec((1, D_H)),
                  _full_spec((D_H, D_H)), _full_spec((D_H, D_H))],
        out_specs=pl.BlockSpec((R, 2 * D_H), lambda i: (i, 0)),
        out_shape=jax.ShapeDtypeStruct((N, 2 * D_H), jnp.float32),
    )(s2, hp2, degp, b_enc2.reshape(1, D_H), W_stru, W_att1)

    s34 = _spmm_sc(hp34, src, dst, 2 * D_H).reshape(2, N, 2 * D_H)

    stru_z, hp5 = pl.pallas_call(
        _t3_body,
        grid=(GRID,),
        in_specs=[_part_spec(2 * D_H), pl.BlockSpec((R, 2 * D_H), lambda i: (i, 0)),
                  _part_spec(16), _full_spec((1, D_H)), _full_spec((1, D_H)),
                  _full_spec((D_H, D_IN))],
        out_specs=_row_specs([D_H, D_IN]),
        out_shape=[jax.ShapeDtypeStruct((N, D_H), jnp.float32),
                   jax.ShapeDtypeStruct((N, D_IN), jnp.float32)],
    )(s34, hp34, degp, b_stru.reshape(1, D_H), b_att1.reshape(1, D_H), W_att2)

    s5 = _spmm_sc(hp5, src, dst, D_IN).reshape(2, N, D_IN)

    attr_recon = pl.pallas_call(
        _t4_body,
        grid=(GRID,),
        in_specs=[_part_spec(D_IN), pl.BlockSpec((R, D_IN), lambda i: (i, 0)),
                  _part_spec(16), _full_spec((1, D_IN))],
        out_specs=pl.BlockSpec((R, D_IN), lambda i: (i, 0)),
        out_shape=jax.ShapeDtypeStruct((N, D_IN), jnp.float32),
    )(s5, hp5, degp, b_att2.reshape(1, D_IN))

    stru_recon = pl.pallas_call(
        _t5_body,
        grid=(GRID, GRID),
        in_specs=[pl.BlockSpec((R, D_H), lambda i, j: (i, 0)),
                  pl.BlockSpec((R, D_H), lambda i, j: (j, 0))],
        out_specs=pl.BlockSpec((R, R), lambda i, j: (i, j)),
        out_shape=jax.ShapeDtypeStruct((N, N), jnp.float32),
    )(stru_z, stru_z)

    return (stru_recon, attr_recon)


# speedup vs baseline: 34.5296x; 34.5296x over previous
import functools

import jax
import jax.numpy as jnp
from jax import lax
from jax.experimental import pallas as pl
from jax.experimental.pallas import tpu as pltpu
from jax.experimental.pallas import tpu_sc as plsc

N = 10000
E = 320000
D_IN = 128
D_H = 64

NC = 2
NS = 16
NW = NC * NS
EPW = E // NW
BATCH = 500
NCHUNK = EPW // BATCH
ROWS_PER_TILE = 624
TAIL = N - NS * ROWS_PER_TILE
ZROWS = 24
NPAIR = N // 2
RP = 1000

def _sc_mesh():
    return plsc.VectorSubcoreMesh(
        core_axis_name="c", subcore_axis_name="s", num_cores=NC, num_subcores=NS)


def _zero_fill(buf, nrows, ncols):
    zero = jnp.zeros((16,), jnp.float32)
    for r in range(nrows):
        for c in range(ncols // 16):
            buf[r, pl.ds(c * 16, 16)] = zero


def _spmm_sc(hp, er, F):
    bat = BATCH
    nch = NCHUNK

    @functools.partial(
        pl.kernel,
        mesh=_sc_mesh(),
        out_type=jax.ShapeDtypeStruct((2, N, F), jnp.float32),
        compiler_params=pltpu.CompilerParams(use_tc_tiling_on_sc=False),
        scratch_types=[
            pltpu.VMEM((nch, bat), jnp.int32),
            pltpu.VMEM((nch, bat), jnp.int32),
            pltpu.VMEM((2, bat, F), jnp.float32),
            pltpu.VMEM((ZROWS, F), jnp.float32),
            pltpu.VMEM_SHARED((N, F), jnp.float32),
            pltpu.SemaphoreType.DMA,
        ],
    )
    def k(hp_hbm, er_hbm, out_hbm, srcb, dstb, rows, zbuf, acc, sem):
        c = lax.axis_index("c")
        s = lax.axis_index("s")
        wid = c * NS + s
        _zero_fill(zbuf, ZROWS, F)
        for i in range(ROWS_PER_TILE // ZROWS):
            pltpu.sync_copy(zbuf, acc.at[pl.ds(s * ROWS_PER_TILE + i * ZROWS, ZROWS)])

        @pl.when(s == 0)
        def _():
            pltpu.sync_copy(zbuf.at[pl.ds(0, TAIL)], acc.at[pl.ds(NS * ROWS_PER_TILE, TAIL)])
        pltpu.sync_copy(er_hbm.at[0, wid], srcb)
        pltpu.sync_copy(er_hbm.at[1, wid], dstb)
        plsc.subcore_barrier()

        pltpu.async_copy(hp_hbm.at[srcb.at[0]], rows.at[0], sem)

        def body(kk, carry):
            j = lax.rem(kk, 2)
            pltpu.make_async_copy(hp_hbm.at[srcb.at[kk]], rows.at[j], sem).wait()
            pltpu.async_copy(hp_hbm.at[srcb.at[kk + 1]], rows.at[1 - j], sem)
            pltpu.sync_copy(rows.at[j], acc.at[dstb.at[kk]], add=True)
            return carry

        lax.fori_loop(0, nch - 1, body, 0)
        jl = (nch - 1) % 2
        pltpu.make_async_copy(
            hp_hbm.at[srcb.at[nch - 1]], rows.at[jl], sem).wait()
        pltpu.sync_copy(rows.at[jl], acc.at[dstb.at[nch - 1]], add=True)
        plsc.subcore_barrier()
        pltpu.sync_copy(
            acc.at[pl.ds(s * ROWS_PER_TILE, ROWS_PER_TILE)],
            out_hbm.at[c, pl.ds(s * ROWS_PER_TILE, ROWS_PER_TILE)],
        )

        @pl.when(s == 0)
        def _():
            pltpu.sync_copy(
                acc.at[pl.ds(NS * ROWS_PER_TILE, TAIL)],
                out_hbm.at[c, pl.ds(NS * ROWS_PER_TILE, TAIL)],
            )

    return k(hp, er).reshape(2, NPAIR, 2 * F)


def _deg_sc(er):
    F = 16

    @functools.partial(
        pl.kernel,
        mesh=_sc_mesh(),
        out_type=jax.ShapeDtypeStruct((2, N, F), jnp.float32),
        compiler_params=pltpu.CompilerParams(use_tc_tiling_on_sc=False),
        scratch_types=[
            pltpu.VMEM((NCHUNK, BATCH), jnp.int32),
            pltpu.VMEM((BATCH, F), jnp.float32),
            pltpu.VMEM((ZROWS, F), jnp.float32),
            pltpu.VMEM_SHARED((N, F), jnp.float32),
        ],
    )
    def k(er_hbm, out_hbm, dstb, ones, zbuf, acc):
        c = lax.axis_index("c")
        s = lax.axis_index("s")
        wid = c * NS + s
        _zero_fill(zbuf, ZROWS, F)
        one = jnp.ones((16,), jnp.float32)
        for r in range(BATCH):
            ones[r, pl.ds(0, 16)] = one
        for i in range(ROWS_PER_TILE // ZROWS):
            pltpu.sync_copy(zbuf, acc.at[pl.ds(s * ROWS_PER_TILE + i * ZROWS, ZROWS)])

        @pl.when(s == 0)
        def _():
            pltpu.sync_copy(zbuf.at[pl.ds(0, TAIL)], acc.at[pl.ds(NS * ROWS_PER_TILE, TAIL)])
        pltpu.sync_copy(er_hbm.at[1, wid], dstb)
        plsc.subcore_barrier()

        def body(kk, carry):
            pltpu.sync_copy(ones, acc.at[dstb.at[kk]], add=True)
            return carry

        lax.fori_loop(0, NCHUNK, body, 0)
        plsc.subcore_barrier()
        pltpu.sync_copy(
            acc.at[pl.ds(s * ROWS_PER_TILE, ROWS_PER_TILE)],
            out_hbm.at[c, pl.ds(s * ROWS_PER_TILE, ROWS_PER_TILE)],
        )

        @pl.when(s == 0)
        def _():
            pltpu.sync_copy(
                acc.at[pl.ds(NS * ROWS_PER_TILE, TAIL)],
                out_hbm.at[c, pl.ds(NS * ROWS_PER_TILE, TAIL)],
            )

    return k(er)


R = 2000
GRID = N // R


def _t00_body(degpp_ref, dinvp_ref):
    b = degpp_ref[...]
    de = lax.rsqrt(b[0, :, 0] + b[1, :, 0] + 1.0)[:, None]
    do = lax.rsqrt(b[0, :, 16] + b[1, :, 16] + 1.0)[:, None]
    dinvp_ref[...] = jnp.concatenate(
        [jnp.broadcast_to(de, (RP, D_H)), jnp.broadcast_to(do, (RP, D_H))],
        axis=1)


def _t0_body(xp_ref, w1bd_ref, dinvp_ref, hp1_ref):
    h = jnp.dot(xp_ref[...], w1bd_ref[...], preferred_element_type=jnp.float32)
    hp1_ref[...] = dinvp_ref[...] * h


def _t1_body(s1_ref, hp1_ref, dinvp_ref, b1p_ref, hz_ref):
    dp = dinvp_ref[...]
    sp = s1_ref[0] + s1_ref[1] + hp1_ref[...]
    z = jnp.maximum(dp * sp + b1p_ref[...], 0.0)
    hz_ref[...] = dp * z


def _t2_body(s2_ref, hz_ref, dinvp_ref, w2bd_ref, b2p_ref, hz2_ref):
    dp = dinvp_ref[...]
    q2 = dp * (s2_ref[0] + s2_ref[1] + hz_ref[...])
    z2 = jnp.maximum(
        jnp.dot(q2, w2bd_ref[...], preferred_element_type=jnp.float32)
        + b2p_ref[...], 0.0)
    hz2_ref[...] = dp * z2


def _t3_body(s3_ref, hz2_ref, dinvp_ref, wsbd_ref, bsp_ref, wa1bd_ref,
             ba1p_ref, struzp_ref, ha_ref):
    dp = dinvp_ref[...]
    q3p = dp * (s3_ref[0] + s3_ref[1] + hz2_ref[...])
    struzp_ref[...] = jnp.maximum(
        jnp.dot(q3p, wsbd_ref[...], preferred_element_type=jnp.float32)
        + bsp_ref[...], 0.0)
    ap = jnp.maximum(
        jnp.dot(q3p, wa1bd_ref[...], preferred_element_type=jnp.float32)
        + ba1p_ref[...], 0.0)
    ha_ref[...] = dp * ap


def _t4_body(s4_ref, ha_ref, dinvp_ref, wa2bd_ref, ba2p_ref, attrp_ref):
    dp = dinvp_ref[...]
    q4p = dp * (s4_ref[0] + s4_ref[1] + ha_ref[...])
    attrp_ref[...] = jnp.maximum(
        jnp.dot(q4p, wa2bd_ref[...], preferred_element_type=jnp.float32)
        + ba2p_ref[...], 0.0)


def _t5_body(zl_ref, zr_ref, out_ref):
    out_ref[...] = lax.dot_general(
        zl_ref[...], zr_ref[...], (((1,), (1,)), ((), ())),
        preferred_element_type=jnp.float32)


def _full_spec(shape):
    return pl.BlockSpec(shape, lambda i: tuple(0 for _ in shape))


def _pair_spec():
    return pl.BlockSpec((RP, 2 * D_H), lambda i: (i, 0))


def _ppart_spec():
    return pl.BlockSpec((2, RP, 2 * D_H), lambda i: (0, i, 0))


def _bdiag(w):
    z = jnp.zeros_like(w)
    return jnp.concatenate(
        [jnp.concatenate([w, z], axis=1), jnp.concatenate([z, w], axis=1)],
        axis=0)


def kernel(x, edge_index, W_enc1, b_enc1, W_enc2, b_enc2, W_stru, b_stru,
           W_att1, b_att1, W_att2, b_att2):
    er = edge_index.reshape(2, NW, NCHUNK, BATCH)
    xp = x.reshape(NPAIR, 2 * D_IN)
    w1bd = _bdiag(W_enc1)
    w2bd = _bdiag(W_enc2)
    wsbd = _bdiag(W_stru)
    wa1bd = _bdiag(W_att1)
    wa2bd = _bdiag(W_att2)
    b1p = jnp.tile(b_enc1, 2).reshape(1, 2 * D_H)
    b2p = jnp.tile(b_enc2, 2).reshape(1, 2 * D_H)
    bsp = jnp.tile(b_stru, 2).reshape(1, 2 * D_H)
    ba1p = jnp.tile(b_att1, 2).reshape(1, 2 * D_H)
    ba2p = jnp.tile(b_att2, 2).reshape(1, 2 * D_IN)

    degp = _deg_sc(er).reshape(2, NPAIR, 32)

    dinvp = pl.pallas_call(
        _t00_body,
        grid=(GRID,),
        in_specs=[pl.BlockSpec((2, RP, 32), lambda i: (0, i, 0))],
        out_specs=_pair_spec(),
        out_shape=jax.ShapeDtypeStruct((NPAIR, 2 * D_H), jnp.float32),
    )(degp)

    hp1p = pl.pallas_call(
        _t0_body,
        grid=(GRID,),
        in_specs=[pl.BlockSpec((RP, 2 * D_IN), lambda i: (i, 0)),
                  _full_spec((2 * D_IN, 2 * D_H)), _pair_spec()],
        out_specs=_pair_spec(),
        out_shape=jax.ShapeDtypeStruct((NPAIR, 2 * D_H), jnp.float32),
    )(xp, w1bd, dinvp)

    s1 = _spmm_sc(hp1p.reshape(N, D_H), er, D_H)

    hzp = pl.pallas_call(
        _t1_body,
        grid=(GRID,),
        in_specs=[_ppart_spec(), _pair_spec(), _pair_spec(),
                  _full_spec((1, 2 * D_H))],
        out_specs=_pair_spec(),
        out_shape=jax.ShapeDtypeStruct((NPAIR, 2 * D_H), jnp.float32),
    )(s1, hp1p, dinvp, b1p)

    s2 = _spmm_sc(hzp.reshape(N, D_H), er, D_H)

    hz2p = pl.pallas_call(
        _t2_body,
        grid=(GRID,),
        in_specs=[_ppart_spec(), _pair_spec(), _pair_spec(),
                  _full_spec((2 * D_H, 2 * D_H)), _full_spec((1, 2 * D_H))],
        out_specs=_pair_spec(),
        out_shape=jax.ShapeDtypeStruct((NPAIR, 2 * D_H), jnp.float32),
    )(s2, hzp, dinvp, w2bd, b2p)

    s3 = _spmm_sc(hz2p.reshape(N, D_H), er, D_H)

    struzp, hap = pl.pallas_call(
        _t3_body,
        grid=(GRID,),
        in_specs=[_ppart_spec(), _pair_spec(), _pair_spec(),
                  _full_spec((2 * D_H, 2 * D_H)), _full_spec((1, 2 * D_H)),
                  _full_spec((2 * D_H, 2 * D_H)), _full_spec((1, 2 * D_H))],
        out_specs=[_pair_spec(), _pair_spec()],
        out_shape=[jax.ShapeDtypeStruct((NPAIR, 2 * D_H), jnp.float32),
                   jax.ShapeDtypeStruct((NPAIR, 2 * D_H), jnp.float32)],
    )(s3, hz2p, dinvp, wsbd, bsp, wa1bd, ba1p)

    stru_z = struzp.reshape(N, D_H)
    R5 = 400
    stru_recon = pl.pallas_call(
        _t5_body,
        grid=(N // R5,),
        in_specs=[pl.BlockSpec((R5, D_H), lambda i: (i, 0)),
                  pl.BlockSpec((N, D_H), lambda i: (0, 0))],
        out_specs=pl.BlockSpec((R5, N), lambda i: (i, 0)),
        out_shape=jax.ShapeDtypeStruct((N, N), jnp.float32),
    )(stru_z, stru_z)

    s4 = _spmm_sc(hap.reshape(N, D_H), er, D_H)

    attrp = pl.pallas_call(
        _t4_body,
        grid=(GRID,),
        in_specs=[_ppart_spec(), _pair_spec(), _pair_spec(),
                  _full_spec((2 * D_H, 2 * D_IN)), _full_spec((1, 2 * D_IN))],
        out_specs=pl.BlockSpec((RP, 2 * D_IN), lambda i: (i, 0)),
        out_shape=jax.ShapeDtypeStruct((NPAIR, 2 * D_IN), jnp.float32),
    )(s4, hap, dinvp, wa2bd, ba2p)

    return (stru_recon, attrp.reshape(N, D_IN))
